# baseline (speedup 1.0000x reference)
import jax
import jax.numpy as jnp
from jax.experimental import pallas as pl
from jax.experimental.pallas import tpu as pltpu

_N = 10000
_D = 256
_C = 64

_TM = 200
_NBLK = _N // _TM


def _gcn_kernel(x_ref, adj_ref, w1_ref, b1_ref, wmt_ref, bm_ref, out_ref,
                sup):
    i = pl.program_id(0)

    @pl.when(i == 0)
    def _():
        sup[...] = jnp.dot(x_ref[...], w1_ref[...],
                           preferred_element_type=jnp.float32)

    h = jnp.dot(adj_ref[...], sup[...],
                preferred_element_type=jnp.float32)
    h = jnp.maximum(h + b1_ref[...], 0.0)
    out_ref[...] = jnp.dot(
        h, wmt_ref[...], preferred_element_type=jnp.float32,
    ) + bm_ref[...]


def kernel(x, adj, W1, b1, W_mlp, b_mlp):
    wmt = W_mlp.T
    b1_2d = b1.reshape(1, _D)
    bm_2d = b_mlp.reshape(1, _C)

    out = pl.pallas_call(
        _gcn_kernel,
        grid=(_NBLK,),
        in_specs=[
            pl.BlockSpec((_N, _D), lambda i: (0, 0)),
            pl.BlockSpec((_TM, _N), lambda i: (i, 0)),
            pl.BlockSpec((_D, _D), lambda i: (0, 0)),
            pl.BlockSpec((1, _D), lambda i: (0, 0)),
            pl.BlockSpec((_D, _C), lambda i: (0, 0)),
            pl.BlockSpec((1, _C), lambda i: (0, 0)),
        ],
        out_specs=pl.BlockSpec((_TM, _C), lambda i: (i, 0)),
        out_shape=jax.ShapeDtypeStruct((_N, _C), jnp.float32),
        scratch_shapes=[
            pltpu.VMEM((_N, _D), jnp.float32),
        ],
        compiler_params=pltpu.CompilerParams(
            dimension_semantics=("arbitrary",),
            vmem_limit_bytes=100 * 1024 * 1024,
        ),
    )(x, adj, W1, b1_2d, wmt, bm_2d)
    return out


# speedup vs baseline: 1.0336x; 1.0336x over previous
import jax
import jax.numpy as jnp
from jax.experimental import pallas as pl
from jax.experimental.pallas import tpu as pltpu

_N = 10000
_D = 256
_C = 64

_TM = 400
_NBLK = _N // _TM


def _gcn_kernel(x_ref, adj_ref, w1_ref, b1_ref, wmt_ref, bm_ref, out_ref,
                sup):
    i = pl.program_id(0)

    @pl.when(i == 0)
    def _():
        sup[...] = jnp.dot(x_ref[...], w1_ref[...],
                           preferred_element_type=jnp.float32
                           ).astype(jnp.bfloat16)

    h = jnp.dot(adj_ref[...].astype(jnp.bfloat16), sup[...],
                preferred_element_type=jnp.float32)
    h = jnp.maximum(h + b1_ref[...], 0.0)
    out_ref[...] = jnp.dot(
        h, wmt_ref[...], preferred_element_type=jnp.float32,
    ) + bm_ref[...]


def kernel(x, adj, W1, b1, W_mlp, b_mlp):
    wmt = W_mlp.T
    b1_2d = b1.reshape(1, _D)
    bm_2d = b_mlp.reshape(1, _C)

    out = pl.pallas_call(
        _gcn_kernel,
        grid=(_NBLK,),
        in_specs=[
            pl.BlockSpec((_N, _D), lambda i: (0, 0)),
            pl.BlockSpec((_TM, _N), lambda i: (i, 0)),
            pl.BlockSpec((_D, _D), lambda i: (0, 0)),
            pl.BlockSpec((1, _D), lambda i: (0, 0)),
            pl.BlockSpec((_D, _C), lambda i: (0, 0)),
            pl.BlockSpec((1, _C), lambda i: (0, 0)),
        ],
        out_specs=pl.BlockSpec((_TM, _C), lambda i: (i, 0)),
        out_shape=jax.ShapeDtypeStruct((_N, _C), jnp.float32),
        scratch_shapes=[
            pltpu.VMEM((_N, _D), jnp.bfloat16),
        ],
        compiler_params=pltpu.CompilerParams(
            dimension_semantics=("arbitrary",),
            vmem_limit_bytes=100 * 1024 * 1024,
        ),
    )(x, adj, W1, b1_2d, wmt, bm_2d)
    return out


# speedup vs baseline: 1.0459x; 1.0119x over previous
import jax
import jax.numpy as jnp
from jax.experimental import pallas as pl
from jax.experimental.pallas import tpu as pltpu

_N = 10000
_D = 256
_C = 64

_TM = 400
_NBLK = _N // _TM


def _gcn_kernel(x_ref, adj_ref, w1_ref, b1_ref, wmt_ref, bm_ref, out_ref,
                sup):
    i = pl.program_id(0)

    @pl.when(i == 0)
    def _():
        sup[...] = jnp.dot(x_ref[...], w1_ref[...],
                           preferred_element_type=jnp.float32
                           ).astype(jnp.bfloat16)

    h = jnp.dot(adj_ref[...].astype(jnp.bfloat16), sup[...],
                preferred_element_type=jnp.float32)
    h = jnp.maximum(h + b1_ref[...], 0.0)
    out_ref[...] = jax.lax.dot_general(
        h, wmt_ref[...], (((1,), (1,)), ((), ())),
        preferred_element_type=jnp.float32,
    ) + bm_ref[...]


def kernel(x, adj, W1, b1, W_mlp, b_mlp):
    b1_2d = b1.reshape(1, _D)
    bm_2d = b_mlp.reshape(1, _C)

    out = pl.pallas_call(
        _gcn_kernel,
        grid=(_NBLK,),
        in_specs=[
            pl.BlockSpec((_N, _D), lambda i: (0, 0)),
            pl.BlockSpec((_TM, _N), lambda i: (i, 0)),
            pl.BlockSpec((_D, _D), lambda i: (0, 0)),
            pl.BlockSpec((1, _D), lambda i: (0, 0)),
            pl.BlockSpec((_C, _D), lambda i: (0, 0)),
            pl.BlockSpec((1, _C), lambda i: (0, 0)),
        ],
        out_specs=pl.BlockSpec((_TM, _C), lambda i: (i, 0)),
        out_shape=jax.ShapeDtypeStruct((_N, _C), jnp.float32),
        scratch_shapes=[
            pltpu.VMEM((_N, _D), jnp.bfloat16),
        ],
        compiler_params=pltpu.CompilerParams(
            dimension_semantics=("arbitrary",),
            vmem_limit_bytes=100 * 1024 * 1024,
        ),
    )(x, adj, W1, b1_2d, W_mlp, bm_2d)
    return out
